# baseline (speedup 1.0000x reference)
import functools

import jax
import jax.numpy as jnp
from jax import lax
from jax.experimental import pallas as pl
from jax.experimental.pallas import tpu as pltpu
from jax.experimental.pallas import tpu_sc as plsc

_B, _T, _Z = 4096, 65, 128
_NW = 32
_RPW = _B // _NW
_L = 16


def _rsqrt_nr(x):
    i = lax.bitcast_convert_type(x, jnp.int32)
    i = jnp.int32(0x5F3759DF) - (i >> 1)
    y = lax.bitcast_convert_type(i, jnp.float32)
    return y * (1.5 - 0.5 * x * y * y)


def _sc_body(dist_hbm, rh_hbm, zv_hbm, out_hbm, dist_v, rh_v, zv_v, out_v):
    c = lax.axis_index("c")
    s = lax.axis_index("s")
    wid = s * 2 + c
    base = wid * _RPW
    pltpu.sync_copy(dist_hbm.at[pl.ds(base * _T, _RPW * _T)], dist_v)
    pltpu.sync_copy(rh_hbm.at[pl.ds(base * (_T * 3), _RPW * _T * 3)], rh_v)
    pltpu.sync_copy(zv_hbm.at[pl.ds(base * _Z, _RPW * _Z)], zv_v)

    lanes3 = lax.iota(jnp.int32, _L) * 3
    nz = _Z // _L

    def ray(r, carry):
        rd = jnp.full((_L,), r * _T, jnp.int32)
        ro = jnp.full((_L,), r * (_Z * 3), jnp.int32)
        z = [zv_v[pl.ds(r * _Z + zi * _L, _L)] for zi in range(nz)]
        lo = [rd for _ in range(nz)]
        for step in (32, 16, 8, 4, 2, 1):
            dp = [plsc.load_gather(dist_v, [lo[zi] + step]) for zi in range(nz)]
            for zi in range(nz):
                lo[zi] = jnp.where(dp[zi] <= z[zi], lo[zi] + step, lo[zi])
        for zi in range(nz):
            d0 = plsc.load_gather(dist_v, [lo[zi]])
            vpos = z[zi] - d0
            b0 = lo[zi] * 3
            c0 = [plsc.load_gather(rh_v, [b0 + k]) for k in range(3)]
            c1 = [plsc.load_gather(rh_v, [b0 + (3 + k)]) for k in range(3)]
            m = [c1[k] - c0[k] for k in range(3)]
            n2 = m[0] * m[0] + m[1] * m[1] + m[2] * m[2]
            scale = vpos * _rsqrt_nr(n2)
            o0 = ro + (zi * (_L * 3)) + lanes3
            for k in range(3):
                plsc.store_scatter(out_v, [o0 + k], c0[k] + scale * m[k])
        return carry

    lax.fori_loop(0, _RPW, ray, 0)
    pltpu.sync_copy(out_v, out_hbm.at[pl.ds(base * (_Z * 3), _RPW * _Z * 3)])


@functools.partial(
    pl.kernel,
    out_type=jax.ShapeDtypeStruct((_B * _Z * 3,), jnp.float32),
    mesh=plsc.VectorSubcoreMesh(core_axis_name="c", subcore_axis_name="s"),
    compiler_params=pltpu.CompilerParams(needs_layout_passes=False),
    scratch_types=[
        pltpu.VMEM((_RPW * _T,), jnp.float32),
        pltpu.VMEM((_RPW * _T * 3,), jnp.float32),
        pltpu.VMEM((_RPW * _Z,), jnp.float32),
        pltpu.VMEM((_RPW * _Z * 3,), jnp.float32),
    ],
)
def _evolution_sc(dist_hbm, rh_hbm, zv_hbm, out_hbm, dist_v, rh_v, zv_v, out_v):
    _sc_body(dist_hbm, rh_hbm, zv_hbm, out_hbm, dist_v, rh_v, zv_v, out_v)


def kernel(r_hist, distances, z_vals):
    zv = z_vals.reshape(_B * _Z)
    rh = r_hist.reshape(_B * _T * 3)
    dist = distances.reshape(_B * _T)
    out = _evolution_sc(dist, rh, zv)
    return out.reshape(_B, _Z, 3)


# speedup vs baseline: 8.2689x; 8.2689x over previous
import functools

import jax
import jax.numpy as jnp
from jax import lax
from jax.experimental import pallas as pl
from jax.experimental.pallas import tpu as pltpu
from jax.experimental.pallas import tpu_sc as plsc

_B, _T, _Z = 4096, 65, 128
_NW = 32
_RPW = _B // _NW
_L = 16


def _rsqrt_nr(x):
    i = lax.bitcast_convert_type(x, jnp.int32)
    i = jnp.int32(0x5F3759DF) - (i >> 1)
    y = lax.bitcast_convert_type(i, jnp.float32)
    return y * (1.5 - 0.5 * x * y * y)


def _sc_body(dist_hbm, rh_hbm, zv_hbm, out_hbm, dist_v, rh_v, zv_v, out_v):
    c = lax.axis_index("c")
    s = lax.axis_index("s")
    wid = s * 2 + c
    base = wid * _RPW
    pltpu.sync_copy(dist_hbm.at[pl.ds(base, _RPW)], dist_v)
    pltpu.sync_copy(rh_hbm.at[pl.ds(base, _RPW)], rh_v)
    pltpu.sync_copy(zv_hbm.at[pl.ds(base, _RPW)], zv_v)

    nz = _Z // _L

    def ray(r, carry):
        r_s = jnp.full((_L,), r, jnp.int32)
        z = [zv_v[r, pl.ds(zi * _L, _L)] for zi in range(nz)]
        lo = [jnp.zeros((_L,), jnp.int32) for _ in range(nz)]
        for step in (32, 16, 8, 4, 2, 1):
            dp = [plsc.load_gather(dist_v, [r_s, lo[zi] + step])
                  for zi in range(nz)]
            for zi in range(nz):
                lo[zi] = jnp.where(dp[zi] <= z[zi], lo[zi] + step, lo[zi])
        for zi in range(nz):
            d0 = plsc.load_gather(dist_v, [r_s, lo[zi]])
            vpos = z[zi] - d0
            b0 = lo[zi] * 3
            c0 = [plsc.load_gather(rh_v, [r_s, b0 + k]) for k in range(3)]
            c1 = [plsc.load_gather(rh_v, [r_s, b0 + (3 + k)]) for k in range(3)]
            m = [c1[k] - c0[k] for k in range(3)]
            n2 = m[0] * m[0] + m[1] * m[1] + m[2] * m[2]
            scale = vpos * _rsqrt_nr(n2)
            for k in range(3):
                out_v[k, r, pl.ds(zi * _L, _L)] = c0[k] + scale * m[k]
        return carry

    lax.fori_loop(0, _RPW, ray, 0)
    for k in range(3):
        pltpu.sync_copy(out_v.at[k], out_hbm.at[k, pl.ds(base, _RPW)])


@functools.partial(
    pl.kernel,
    out_type=jax.ShapeDtypeStruct((3, _B, _Z), jnp.float32),
    mesh=plsc.VectorSubcoreMesh(core_axis_name="c", subcore_axis_name="s"),
    compiler_params=pltpu.CompilerParams(needs_layout_passes=False),
    scratch_types=[
        pltpu.VMEM((_RPW, _T), jnp.float32),
        pltpu.VMEM((_RPW, _T * 3), jnp.float32),
        pltpu.VMEM((_RPW, _Z), jnp.float32),
        pltpu.VMEM((3, _RPW, _Z), jnp.float32),
    ],
)
def _evolution_sc(dist_hbm, rh_hbm, zv_hbm, out_hbm, dist_v, rh_v, zv_v, out_v):
    _sc_body(dist_hbm, rh_hbm, zv_hbm, out_hbm, dist_v, rh_v, zv_v, out_v)


def kernel(r_hist, distances, z_vals):
    zv = z_vals.reshape(_B, _Z)
    rh = r_hist.reshape(_B, _T * 3)
    out = _evolution_sc(distances, rh, zv)
    return out.transpose(1, 2, 0)


# speedup vs baseline: 9.4798x; 1.1464x over previous
import functools

import jax
import jax.numpy as jnp
from jax import lax
from jax.experimental import pallas as pl
from jax.experimental.pallas import tpu as pltpu
from jax.experimental.pallas import tpu_sc as plsc

_B, _T, _Z = 4096, 65, 128
_NW = 32
_RPW = _B // _NW
_L = 16


def _rsqrt_nr(x):
    i = lax.bitcast_convert_type(x, jnp.int32)
    i = jnp.int32(0x5F3759DF) - (i >> 1)
    y = lax.bitcast_convert_type(i, jnp.float32)
    return y * (1.5 - 0.5 * x * y * y)


def _sc_body(dist_hbm, rx_hbm, ry_hbm, rz_hbm, zv_hbm, out_hbm,
             dist_v, rx_v, ry_v, rz_v, zv_v, out_v, sem):
    c = lax.axis_index("c")
    s = lax.axis_index("s")
    wid = s * 2 + c
    base = wid * _RPW
    sl = pl.ds(base, _RPW)
    cps = [pltpu.async_copy(src.at[sl], dst, sem)
           for src, dst in ((dist_hbm, dist_v), (rx_hbm, rx_v),
                            (ry_hbm, ry_v), (rz_hbm, rz_v), (zv_hbm, zv_v))]
    for cp in cps:
        cp.wait()

    nz = _Z // _L
    planes = (rx_v, ry_v, rz_v)

    def ray(r, carry):
        r_s = jnp.full((_L,), r, jnp.int32)
        z = [zv_v[r, pl.ds(zi * _L, _L)] for zi in range(nz)]
        lo = [jnp.zeros((_L,), jnp.int32) for _ in range(nz)]
        for step in (32, 16, 8, 4, 2, 1):
            dp = [plsc.load_gather(dist_v, [r_s, lo[zi] + step])
                  for zi in range(nz)]
            for zi in range(nz):
                lo[zi] = jnp.where(dp[zi] <= z[zi], lo[zi] + step, lo[zi])
        for zi in range(nz):
            d0 = plsc.load_gather(dist_v, [r_s, lo[zi]])
            vpos = z[zi] - d0
            hi = lo[zi] + 1
            c0 = [plsc.load_gather(planes[k], [r_s, lo[zi]]) for k in range(3)]
            c1 = [plsc.load_gather(planes[k], [r_s, hi]) for k in range(3)]
            m = [c1[k] - c0[k] for k in range(3)]
            n2 = m[0] * m[0] + m[1] * m[1] + m[2] * m[2]
            scale = vpos * _rsqrt_nr(n2)
            for k in range(3):
                out_v[k, r, pl.ds(zi * _L, _L)] = c0[k] + scale * m[k]
        return carry

    lax.fori_loop(0, _RPW, ray, 0)
    for k in range(3):
        pltpu.sync_copy(out_v.at[k], out_hbm.at[k, sl])


@functools.partial(
    pl.kernel,
    out_type=jax.ShapeDtypeStruct((3, _B, _Z), jnp.float32),
    mesh=plsc.VectorSubcoreMesh(core_axis_name="c", subcore_axis_name="s"),
    compiler_params=pltpu.CompilerParams(needs_layout_passes=False),
    scratch_types=[
        pltpu.VMEM((_RPW, _T), jnp.float32),
        pltpu.VMEM((_RPW, _T), jnp.float32),
        pltpu.VMEM((_RPW, _T), jnp.float32),
        pltpu.VMEM((_RPW, _T), jnp.float32),
        pltpu.VMEM((_RPW, _Z), jnp.float32),
        pltpu.VMEM((3, _RPW, _Z), jnp.float32),
        pltpu.SemaphoreType.DMA,
    ],
)
def _evolution_sc(dist_hbm, rx_hbm, ry_hbm, rz_hbm, zv_hbm, out_hbm,
                  dist_v, rx_v, ry_v, rz_v, zv_v, out_v, sem):
    _sc_body(dist_hbm, rx_hbm, ry_hbm, rz_hbm, zv_hbm, out_hbm,
             dist_v, rx_v, ry_v, rz_v, zv_v, out_v, sem)


def kernel(r_hist, distances, z_vals):
    zv = z_vals.reshape(_B, _Z)
    rx = r_hist[:, :, 0]
    ry = r_hist[:, :, 1]
    rz = r_hist[:, :, 2]
    out = _evolution_sc(distances, rx, ry, rz, zv)
    return out.transpose(1, 2, 0)
